# Initial kernel scaffold; baseline (speedup 1.0000x reference)
import jax
import jax.numpy as jnp
from jax.experimental import pallas as pl


def kernel(x, weight, w_ih, w_hh, b_ih, b_hh, edge_index, mapping_idx):
    raise NotImplementedError("write your pallas kernel here")


# speedup vs baseline: 8.1084x; 8.1084x over previous
import functools

import jax
import jax.numpy as jnp
from jax import lax
from jax.experimental import pallas as pl
from jax.experimental.pallas import tpu as pltpu
from jax.experimental.pallas import tpu_sc as plsc

N_LAYERS = 3
NC, NS = 2, 16
NW = NC * NS
CHUNK = 125


def _mm_body(h_ref, w_ref, o_ref):
    o_ref[...] = jnp.dot(h_ref[...], w_ref[...],
                         preferred_element_type=jnp.float32)


def _matmul(h, w, br):
    n, d = h.shape
    return pl.pallas_call(
        _mm_body,
        grid=(n // br,),
        in_specs=[pl.BlockSpec((br, d), lambda i: (i, 0)),
                  pl.BlockSpec((d, d), lambda i: (0, 0))],
        out_specs=pl.BlockSpec((br, d), lambda i: (i, 0)),
        out_shape=jax.ShapeDtypeStruct((n, d), jnp.float32),
    )(h, w)


def _gru_body(p_ref, h_ref, wih_ref, whh_ref, bih_ref, bhh_ref, o_ref):
    d = h_ref.shape[1]
    agg = p_ref[0] + p_ref[1]
    h = h_ref[...]
    gi = jnp.dot(agg, wih_ref[...], preferred_element_type=jnp.float32)
    gi = gi + bih_ref[...]
    gh = jnp.dot(h, whh_ref[...], preferred_element_type=jnp.float32)
    gh = gh + bhh_ref[...]
    r = jax.nn.sigmoid(gi[:, :d] + gh[:, :d])
    z = jax.nn.sigmoid(gi[:, d:2 * d] + gh[:, d:2 * d])
    n = jnp.tanh(gi[:, 2 * d:] + r * gh[:, 2 * d:])
    o_ref[...] = (1.0 - z) * n + z * h


def _gru(partial, h, wih_t, whh_t, bih, bhh, br):
    n, d = h.shape
    return pl.pallas_call(
        _gru_body,
        grid=(n // br,),
        in_specs=[pl.BlockSpec((2, br, d), lambda i: (0, i, 0)),
                  pl.BlockSpec((br, d), lambda i: (i, 0)),
                  pl.BlockSpec((d, 3 * d), lambda i: (0, 0)),
                  pl.BlockSpec((d, 3 * d), lambda i: (0, 0)),
                  pl.BlockSpec((1, 3 * d), lambda i: (0, 0)),
                  pl.BlockSpec((1, 3 * d), lambda i: (0, 0))],
        out_specs=pl.BlockSpec((br, d), lambda i: (i, 0)),
        out_shape=jax.ShapeDtypeStruct((n, d), jnp.float32),
    )(partial, h, wih_t, whh_t, bih, bhh)


ZROWS = 104


def _seg_sum_body(n_nodes, d, nchunk, base, rem,
                  m_hbm, src_hbm, dst_hbm, out_hbm,
                  src_v, dst_v, rows_v, zbuf, acc, sem):
    c = lax.axis_index("c")
    s = lax.axis_index("s")
    wid = c * NS + s

    pltpu.sync_copy(src_hbm.at[wid], src_v)
    pltpu.sync_copy(dst_hbm.at[wid], dst_v)

    zeros16 = jnp.zeros((16,), jnp.float32)

    def zrow(i, carry):
        def zcol(j, carry2):
            zbuf[i, pl.ds(j * 16, 16)] = zeros16
            return carry2
        return lax.fori_loop(0, d // 16, zcol, carry)

    lax.fori_loop(0, ZROWS, zrow, 0)
    for k in range(base // ZROWS):
        pltpu.sync_copy(zbuf, acc.at[pl.ds(s * base + k * ZROWS, ZROWS)])

    @pl.when(s == NS - 1)
    def _():
        pltpu.sync_copy(zbuf.at[pl.ds(0, rem)],
                        acc.at[pl.ds(NS * base, rem)])

    plsc.subcore_barrier()

    def chunk_body(j, carry):
        pltpu.async_copy(m_hbm.at[src_v.at[j]], rows_v, sem).wait()
        pltpu.sync_copy(rows_v, acc.at[dst_v.at[j]], add=True)
        return carry

    lax.fori_loop(0, nchunk, chunk_body, 0)
    plsc.subcore_barrier()

    pltpu.sync_copy(acc.at[pl.ds(s * base, base)],
                    out_hbm.at[c, pl.ds(s * base, base)])

    @pl.when(s == NS - 1)
    def _():
        pltpu.sync_copy(acc.at[pl.ds(NS * base, rem)],
                        out_hbm.at[c, pl.ds(NS * base, rem)])


def _seg_sum(m, src_r, dst_r):
    n_nodes, d = m.shape
    nchunk = src_r.shape[1]
    base = (n_nodes // (NS * 8)) * 8
    rem = n_nodes - NS * base
    assert base % ZROWS == 0 and rem <= ZROWS
    mesh = plsc.VectorSubcoreMesh(core_axis_name="c", subcore_axis_name="s")
    f = pl.kernel(
        functools.partial(_seg_sum_body, n_nodes, d, nchunk, base, rem),
        out_type=jax.ShapeDtypeStruct((NC, n_nodes, d), jnp.float32),
        mesh=mesh,
        scratch_types=[
            pltpu.VMEM((nchunk, CHUNK), jnp.int32),
            pltpu.VMEM((nchunk, CHUNK), jnp.int32),
            pltpu.VMEM((CHUNK, d), jnp.float32),
            pltpu.VMEM((ZROWS, d), jnp.float32),
            pltpu.VMEM_SHARED((n_nodes, d), jnp.float32),
            pltpu.SemaphoreType.DMA,
        ],
    )
    return f(m, src_r, dst_r)


def _gather_body(h_hbm, map_hbm, out_hbm, idx_v, rows_v, sem):
    c = lax.axis_index("c")
    s = lax.axis_index("s")
    wid = c * NS + s
    pltpu.sync_copy(map_hbm.at[wid], idx_v)
    pltpu.async_copy(h_hbm.at[idx_v], rows_v, sem).wait()
    pltpu.sync_copy(rows_v, out_hbm.at[wid])


def _gather_clicked(h, mapping_idx):
    batch, num_clicked = mapping_idx.shape
    d = h.shape[1]
    mesh = plsc.VectorSubcoreMesh(core_axis_name="c", subcore_axis_name="s")
    f = pl.kernel(
        _gather_body,
        out_type=jax.ShapeDtypeStruct((batch, num_clicked, d), jnp.float32),
        mesh=mesh,
        scratch_types=[
            pltpu.VMEM((num_clicked,), jnp.int32),
            pltpu.VMEM((num_clicked, d), jnp.float32),
            pltpu.SemaphoreType.DMA,
        ],
    )
    return f(h, mapping_idx)


@jax.jit
def kernel(x, weight, w_ih, w_hh, b_ih, b_hh, edge_index, mapping_idx):
    n_nodes, d = x.shape
    n_edges = edge_index.shape[1]
    ept = n_edges // NW
    nchunk = ept // CHUNK

    src_r = edge_index[0].reshape(NW, nchunk, CHUNK)
    dst_r = edge_index[1].reshape(NW, nchunk, CHUNK)
    wih_t = w_ih.T
    whh_t = w_hh.T
    bih = b_ih.reshape(1, 3 * d)
    bhh = b_hh.reshape(1, 3 * d)

    br = 2000
    h = x
    for i in range(N_LAYERS):
        m = _matmul(h, weight[i], br)
        partial = _seg_sum(m, src_r, dst_r)
        h = _gru(partial, h, wih_t, whh_t, bih, bhh, br)
    return _gather_clicked(h, mapping_idx)
